# baseline (speedup 1.0000x reference)
import functools
import math

import numpy as np
import jax
import jax.numpy as jnp
from jax.experimental import pallas as pl
from jax.experimental.pallas import tpu as pltpu

_EPS = 1e-5


def _round_up(v, m):
    return ((v + m - 1) // m) * m


def _width_matrix_np(n_in, n_out):
    if n_in == 1:
        return np.ones((1, n_out), np.float32)
    src = np.arange(n_out, dtype=np.float64) * (n_in - 1) / (n_out - 1)
    grid = np.arange(n_in, dtype=np.float64)
    m = np.maximum(0.0, 1.0 - np.abs(src[None, :] - grid[:, None]))
    return m.astype(np.float32)


def _height_taps(n_in, n_out):
    taps = []
    for dst in range(n_out):
        if n_in == 1:
            taps.append((0, 0, 1.0, 0.0))
            continue
        src = dst * (n_in - 1) / (n_out - 1)
        i0 = min(int(math.floor(src)), n_in - 1)
        i1 = min(i0 + 1, n_in - 1)
        frac = src - i0
        taps.append((i0, i1, 1.0 - frac, float(frac)))
    return tuple(taps)


def _fused_kernel(aw_ref, w_ref, g_ref, b_ref, x_ref, s_ref, o_ref,
                  t1_s, img_s, rs_s, y_s, sum_s, ssq_s,
                  *, h_taps, kw, h2, w2, cin_p, cout, nb1, nb2, nsteps1,
                  cnt, eps):
    p = h2 * w2
    c2 = 2 * cin_p
    f32 = jnp.float32
    bf16 = jnp.bfloat16
    i = pl.program_id(0)

    lin = jax.lax.broadcasted_iota(jnp.int32, (1, p), 1)
    wpos = lin - (lin // w2) * w2
    zero = jnp.zeros((), bf16)

    pad = _round_up(w2, 128)

    @pl.when(i == 0)
    def _init():
        sum_s[...] = jnp.zeros_like(sum_s)
        ssq_s[...] = jnp.zeros_like(ssq_s)
        rs_s[...] = jnp.zeros_like(rs_s)

    @pl.when(i < nsteps1)
    def _phase1():
        acc_sum = sum_s[...]
        acc_ssq = ssq_s[...]
        for bb in range(nb1):
            t1_s[bb] = jnp.dot(x_ref[bb], aw_ref[...],
                               preferred_element_type=f32)

            for t in range(h2 // 2):
                halves = []
                for hh in (2 * t, 2 * t + 1):
                    i0, i1, a0, a1 = h_taps[hh]
                    r = a0 * t1_s[bb, i0 * cin_p:(i0 + 1) * cin_p, :]
                    if a1 != 0.0:
                        r = r + a1 * t1_s[bb, i1 * cin_p:(i1 + 1) * cin_p, :]
                    halves.append(r)
                img_s[bb, c2:c2 + cin_p, 2 * t * w2:(2 * t + 2) * w2] = (
                    jnp.concatenate(halves, axis=1).astype(bf16))

            img_s[bb, c2 + cin_p:2 * c2, :] = s_ref[bb].astype(bf16)

            c_blk = img_s[bb, c2:2 * c2, :]
            img_s[bb, 0:c2, :] = jnp.where(wpos >= 1,
                                           jnp.roll(c_blk, 1, axis=1), zero)
            img_s[bb, 2 * c2:3 * c2, :] = jnp.where(
                wpos <= w2 - 2, jnp.roll(c_blk, -1, axis=1), zero)

            img = img_s[bb]
            for ky in range(3):
                wk = w_ref[ky].reshape(kw * c2, cout).astype(bf16)
                rs_s[bb, ky * cout:(ky + 1) * cout, pad:pad + p] = (
                    jax.lax.dot_general(
                        wk, img, (((0,), (0,)), ((), ())),
                        preferred_element_type=f32).astype(bf16))

            ch = 512 if p % 512 == 0 else p
            yi = i * nb1 + bb
            for a in range(0, p, ch):
                top = rs_s[bb, 0:cout,
                           pad + a - w2:pad + a - w2 + ch].astype(f32)
                mid = rs_s[bb, cout:2 * cout, pad + a:pad + a + ch].astype(f32)
                bot = rs_s[bb, 2 * cout:3 * cout,
                           pad + a + w2:pad + a + w2 + ch].astype(f32)
                y = jnp.maximum(mid + top + bot, 0.0)
                acc_sum = acc_sum + jnp.sum(y, axis=1, keepdims=True)
                acc_ssq = acc_ssq + jnp.sum(y * y, axis=1, keepdims=True)
                y_s[yi, :, a:a + ch] = y.astype(bf16)
        sum_s[...] = acc_sum
        ssq_s[...] = acc_ssq

    @pl.when(i >= nsteps1)
    def _phase2():
        mean = sum_s[...] * (1.0 / cnt)
        var = jnp.maximum(ssq_s[...] * (1.0 / cnt) - mean * mean, 0.0)
        scale = g_ref[...] * jax.lax.rsqrt(var + eps)
        shift = b_ref[...] - mean * scale
        j = i - nsteps1
        for bb in range(nb2):
            o_ref[bb] = y_s[j * nb2 + bb].astype(f32) * scale + shift


def kernel(x_nchw, skip_nchw, w_hwio, gamma, beta):
    n, cin, h, w = x_nchw.shape
    _, cin_s, h2, w2 = skip_nchw.shape
    kh, kw, cin2, cout = w_hwio.shape
    assert (h2, w2) == (2 * h, 2 * w) and cin_s == cin and cin2 == 2 * cin
    assert kh == 3 and kw == 3
    p = h2 * w2
    cin_p = _round_up(cin, 8)
    c2 = 2 * cin_p
    f32 = jnp.float32
    bf16 = jnp.bfloat16

    aw = jnp.asarray(_width_matrix_np(w, w2), dtype=bf16)
    h_taps = _height_taps(h, h2)

    xp = x_nchw
    sp = skip_nchw
    wq = w_hwio
    if cin_p != cin:
        xp = jnp.pad(xp, ((0, 0), (0, cin_p - cin), (0, 0), (0, 0)))
        sp = jnp.pad(sp, ((0, 0), (0, cin_p - cin), (0, 0), (0, 0)))
        wq = jnp.concatenate(
            [jnp.pad(w_hwio[:, :, :cin, :],
                     ((0, 0), (0, 0), (0, cin_p - cin), (0, 0))),
             jnp.pad(w_hwio[:, :, cin:, :],
                     ((0, 0), (0, 0), (0, cin_p - cin), (0, 0)))], axis=2)
    x2d = jnp.transpose(xp, (0, 2, 1, 3)).reshape(n, h * cin_p, w)
    x2d = x2d.astype(bf16)
    s_flat = sp.reshape(n, cin_p, p)

    nb1 = 2 if n % 2 == 0 else 1
    nb2 = 2 if n % 2 == 0 else 1
    nsteps1 = n // nb1
    body = functools.partial(_fused_kernel, h_taps=h_taps, kw=kw, h2=h2,
                             w2=w2, cin_p=cin_p, cout=cout, nb1=nb1, nb2=nb2,
                             nsteps1=nsteps1, cnt=float(n * p), eps=_EPS)

    last = nsteps1 - 1
    out_flat = pl.pallas_call(
        body,
        out_shape=jax.ShapeDtypeStruct((n, cout, p), f32),
        grid=(nsteps1 + n // nb2,),
        in_specs=[
            pl.BlockSpec((w, w2), lambda i: (0, 0)),
            pl.BlockSpec((kh, kw, c2, cout), lambda i: (0, 0, 0, 0)),
            pl.BlockSpec((cout, 1), lambda i: (0, 0)),
            pl.BlockSpec((cout, 1), lambda i: (0, 0)),
            pl.BlockSpec((nb1, h * cin_p, w),
                         lambda i: (jnp.minimum(i, last), 0, 0)),
            pl.BlockSpec((nb1, cin_p, p),
                         lambda i: (jnp.minimum(i, last), 0, 0)),
        ],
        out_specs=pl.BlockSpec(
            (nb2, cout, p),
            lambda i: (jnp.maximum(i - nsteps1, 0), 0, 0)),
        scratch_shapes=[
            pltpu.VMEM((nb1, h * cin_p, w2), f32),
            pltpu.VMEM((nb1, 3 * c2, p), bf16),
            pltpu.VMEM((nb1, 3 * cout, 2 * _round_up(w2, 128) + p),
                       bf16),
            pltpu.VMEM((n, cout, p), bf16),
            pltpu.VMEM((cout, 1), f32),
            pltpu.VMEM((cout, 1), f32),
        ],
        compiler_params=pltpu.CompilerParams(
            dimension_semantics=("arbitrary",)),
    )(aw, wq, gamma.reshape(cout, 1).astype(f32),
      beta.reshape(cout, 1).astype(f32), x2d, s_flat)

    return out_flat.reshape(n, cout, h2, w2).astype(x_nchw.dtype)


# speedup vs baseline: 1.1555x; 1.1555x over previous
import functools
import math

import numpy as np
import jax
import jax.numpy as jnp
from jax.experimental import pallas as pl
from jax.experimental.pallas import tpu as pltpu

_EPS = 1e-5


def _round_up(v, m):
    return ((v + m - 1) // m) * m


def _width_matrix_np(n_in, n_out):
    if n_in == 1:
        return np.ones((1, n_out), np.float32)
    src = np.arange(n_out, dtype=np.float64) * (n_in - 1) / (n_out - 1)
    grid = np.arange(n_in, dtype=np.float64)
    m = np.maximum(0.0, 1.0 - np.abs(src[None, :] - grid[:, None]))
    return m.astype(np.float32)


def _height_taps(n_in, n_out):
    taps = []
    for dst in range(n_out):
        if n_in == 1:
            taps.append((0, 0, 1.0, 0.0))
            continue
        src = dst * (n_in - 1) / (n_out - 1)
        i0 = min(int(math.floor(src)), n_in - 1)
        i1 = min(i0 + 1, n_in - 1)
        frac = src - i0
        taps.append((i0, i1, 1.0 - frac, float(frac)))
    return tuple(taps)


def _fused_kernel(aw_ref, w_ref, g_ref, b_ref, x_ref, s_ref, o_ref,
                  t1_s, img_s, rs_s, y_s, sum_s, ssq_s,
                  *, h_taps, kw, h2, w2, cin_p, cout, nb1, nb2, nsteps1,
                  cnt, eps):
    p = h2 * w2
    c2 = 2 * cin_p
    f32 = jnp.float32
    bf16 = jnp.bfloat16
    i = pl.program_id(0)

    lin = jax.lax.broadcasted_iota(jnp.int32, (1, p), 1)
    wpos = lin - (lin // w2) * w2
    zero = jnp.zeros((), bf16)

    @pl.when(i == 0)
    def _init():
        sum_s[...] = jnp.zeros_like(sum_s)
        ssq_s[...] = jnp.zeros_like(ssq_s)

    @pl.when(i < nsteps1)
    def _phase1():
        acc_sum = sum_s[...]
        acc_ssq = ssq_s[...]
        for bb in range(nb1):
            t1_s[bb] = jnp.dot(x_ref[bb], aw_ref[...],
                               preferred_element_type=f32)

            for t in range(h2 // 2):
                halves = []
                for hh in (2 * t, 2 * t + 1):
                    i0, i1, a0, a1 = h_taps[hh]
                    r = a0 * t1_s[bb, i0 * cin_p:(i0 + 1) * cin_p, :]
                    if a1 != 0.0:
                        r = r + a1 * t1_s[bb, i1 * cin_p:(i1 + 1) * cin_p, :]
                    halves.append(r)
                img_s[bb, c2:c2 + cin_p, 2 * t * w2:(2 * t + 2) * w2] = (
                    jnp.concatenate(halves, axis=1).astype(bf16))

            img_s[bb, c2 + cin_p:2 * c2, :] = s_ref[bb].astype(bf16)

            c_blk = img_s[bb, c2:2 * c2, :]
            img_s[bb, 0:c2, :] = jnp.where(wpos >= 1,
                                           jnp.roll(c_blk, 1, axis=1), zero)
            img_s[bb, 2 * c2:3 * c2, :] = jnp.where(
                wpos <= w2 - 2, jnp.roll(c_blk, -1, axis=1), zero)

            img = img_s[bb]
            for ki, ky in enumerate((0, 2)):
                wk = w_ref[ky].reshape(kw * c2, cout).astype(bf16)
                rs_s[bb, ki * cout:(ki + 1) * cout, :] = jax.lax.dot_general(
                    wk, img, (((0,), (0,)), ((), ())),
                    preferred_element_type=f32).astype(bf16)
            wk = w_ref[1].reshape(kw * c2, cout).astype(bf16)
            mid = jax.lax.dot_general(wk, img, (((0,), (0,)), ((), ())),
                                      preferred_element_type=f32)

            top = jnp.where(lin >= w2,
                            jnp.roll(rs_s[bb, 0:cout, :], w2, axis=1),
                            zero).astype(f32)
            bot = jnp.where(lin < p - w2,
                            jnp.roll(rs_s[bb, cout:2 * cout, :], -w2,
                                     axis=1), zero).astype(f32)
            y = jnp.maximum(mid + top + bot, 0.0)

            acc_sum = acc_sum + jnp.sum(y, axis=1, keepdims=True)
            acc_ssq = acc_ssq + jnp.sum(y * y, axis=1, keepdims=True)
            y_s[i * nb1 + bb] = y.astype(bf16)
        sum_s[...] = acc_sum
        ssq_s[...] = acc_ssq

    @pl.when(i >= nsteps1)
    def _phase2():
        mean = sum_s[...] * (1.0 / cnt)
        var = jnp.maximum(ssq_s[...] * (1.0 / cnt) - mean * mean, 0.0)
        scale = g_ref[...] * jax.lax.rsqrt(var + eps)
        shift = b_ref[...] - mean * scale
        j = i - nsteps1
        for bb in range(nb2):
            o_ref[bb] = y_s[j * nb2 + bb].astype(f32) * scale + shift


def kernel(x_nchw, skip_nchw, w_hwio, gamma, beta):
    n, cin, h, w = x_nchw.shape
    _, cin_s, h2, w2 = skip_nchw.shape
    kh, kw, cin2, cout = w_hwio.shape
    assert (h2, w2) == (2 * h, 2 * w) and cin_s == cin and cin2 == 2 * cin
    assert kh == 3 and kw == 3
    p = h2 * w2
    cin_p = _round_up(cin, 8)
    c2 = 2 * cin_p
    f32 = jnp.float32
    bf16 = jnp.bfloat16

    aw = jnp.asarray(_width_matrix_np(w, w2), dtype=bf16)
    h_taps = _height_taps(h, h2)

    xp = x_nchw
    sp = skip_nchw
    wq = w_hwio
    if cin_p != cin:
        xp = jnp.pad(xp, ((0, 0), (0, cin_p - cin), (0, 0), (0, 0)))
        sp = jnp.pad(sp, ((0, 0), (0, cin_p - cin), (0, 0), (0, 0)))
        wq = jnp.concatenate(
            [jnp.pad(w_hwio[:, :, :cin, :],
                     ((0, 0), (0, 0), (0, cin_p - cin), (0, 0))),
             jnp.pad(w_hwio[:, :, cin:, :],
                     ((0, 0), (0, 0), (0, cin_p - cin), (0, 0)))], axis=2)
    x2d = jnp.transpose(xp, (0, 2, 1, 3)).reshape(n, h * cin_p, w)
    x2d = x2d.astype(bf16)
    s_flat = sp.reshape(n, cin_p, p)

    nb1 = 2 if n % 2 == 0 else 1
    nb2 = 2 if n % 2 == 0 else 1
    nsteps1 = n // nb1
    body = functools.partial(_fused_kernel, h_taps=h_taps, kw=kw, h2=h2,
                             w2=w2, cin_p=cin_p, cout=cout, nb1=nb1, nb2=nb2,
                             nsteps1=nsteps1, cnt=float(n * p), eps=_EPS)

    last = nsteps1 - 1
    out_flat = pl.pallas_call(
        body,
        out_shape=jax.ShapeDtypeStruct((n, cout, p), f32),
        grid=(nsteps1 + n // nb2,),
        in_specs=[
            pl.BlockSpec((w, w2), lambda i: (0, 0)),
            pl.BlockSpec((kh, kw, c2, cout), lambda i: (0, 0, 0, 0)),
            pl.BlockSpec((cout, 1), lambda i: (0, 0)),
            pl.BlockSpec((cout, 1), lambda i: (0, 0)),
            pl.BlockSpec((nb1, h * cin_p, w),
                         lambda i: (jnp.minimum(i, last), 0, 0)),
            pl.BlockSpec((nb1, cin_p, p),
                         lambda i: (jnp.minimum(i, last), 0, 0)),
        ],
        out_specs=pl.BlockSpec(
            (nb2, cout, p),
            lambda i: (jnp.maximum(i - nsteps1, 0), 0, 0)),
        scratch_shapes=[
            pltpu.VMEM((nb1, h * cin_p, w2), f32),
            pltpu.VMEM((nb1, 3 * c2, p), bf16),
            pltpu.VMEM((nb1, 2 * cout, p), bf16),
            pltpu.VMEM((n, cout, p), bf16),
            pltpu.VMEM((cout, 1), f32),
            pltpu.VMEM((cout, 1), f32),
        ],
        compiler_params=pltpu.CompilerParams(
            dimension_semantics=("arbitrary",)),
    )(aw, wq, gamma.reshape(cout, 1).astype(f32),
      beta.reshape(cout, 1).astype(f32), x2d, s_flat)

    return out_flat.reshape(n, cout, h2, w2).astype(x_nchw.dtype)
